# Initial kernel scaffold; baseline (speedup 1.0000x reference)
import jax
import jax.numpy as jnp
from jax.experimental import pallas as pl


def kernel(x_nchw, w1blk, b1blk, w2blk, b2blk, wf1, bf1, wf2, bf2, wf3, bf3):
    raise NotImplementedError("write your pallas kernel here")


# speedup vs baseline: 2.2460x; 2.2460x over previous
import jax
import jax.numpy as jnp
from jax.experimental import pallas as pl
from jax.experimental.pallas import tpu as pltpu

_NBLK = 256


def _lenet_block(x4_ref, w1_ref, b1_ref, w2_ref, b2_ref,
                 wf1_ref, bf1_ref, wf2_ref, bf2_ref, wf3_ref, bf3_ref,
                 out_ref):
    x4 = x4_ref[...]
    nb = x4.shape[2]

    p1 = jnp.concatenate(
        [x4[u:u + 6, v:v + 6] for u in range(2) for v in range(2)],
        axis=-1).reshape(36 * nb, 64)
    y1 = jnp.dot(p1, w1_ref[...], preferred_element_type=jnp.float32)
    y1 = y1 + b1_ref[...]

    groups = []
    for g in range(4):
        q = y1[:, 24 * g:24 * (g + 1)]
        groups.append(jnp.maximum(jnp.maximum(q[:, 0:6], q[:, 6:12]),
                                  jnp.maximum(q[:, 12:18], q[:, 18:24])))
    s1 = jnp.maximum(jnp.concatenate(groups, axis=-1), 0.0)
    s1 = s1.reshape(6, 6, nb, 24)

    p2 = jnp.concatenate(
        [s1[u:u + 4, v:v + 4] for u in range(3) for v in range(3)],
        axis=-1).reshape(16 * nb, 216)
    y2 = jnp.dot(p2, w2_ref[...], preferred_element_type=jnp.float32)
    y2 = y2 + b2_ref[...]

    s2 = jnp.maximum(
        jnp.maximum(jnp.maximum(y2[:, 0:16], y2[:, 16:32]),
                    jnp.maximum(y2[:, 32:48], y2[:, 48:64])), 0.0)

    s2 = s2.reshape(16, nb, 16)
    feats = jnp.concatenate([s2[s] for s in range(16)], axis=-1)

    h1 = jnp.dot(feats, wf1_ref[...], preferred_element_type=jnp.float32)
    h1 = jnp.maximum(h1 + bf1_ref[...], 0.0)
    h2 = jnp.dot(h1, wf2_ref[...], preferred_element_type=jnp.float32)
    h2 = jnp.maximum(h2 + bf2_ref[...], 0.0)
    out_ref[...] = (
        jnp.dot(h2, wf3_ref[...], preferred_element_type=jnp.float32)
        + bf3_ref[...])


def kernel(x_nchw, w1blk, b1blk, w2blk, b2blk, wf1, bf1, wf2, bf2, wf3, bf3):
    n = x_nchw.shape[0]
    nb = max(_NBLK, ((n + _NBLK - 1) // _NBLK) * _NBLK)
    x = x_nchw.astype(jnp.float32).reshape(n, 28, 28)
    if nb != n:
        x = jnp.pad(x, ((0, nb - n), (0, 0), (0, 0)))
    x4 = jnp.transpose(x.reshape(nb, 7, 4, 7, 4),
                       (1, 3, 0, 2, 4)).reshape(7, 7, nb, 16)

    def full(shape):
        return pl.BlockSpec(shape, lambda i, _s=shape: (0,) * len(_s))

    out = pl.pallas_call(
        _lenet_block,
        out_shape=jax.ShapeDtypeStruct((nb, 128), jnp.float32),
        grid=(nb // _NBLK,),
        in_specs=[
            pl.BlockSpec((7, 7, _NBLK, 16), lambda i: (0, 0, i, 0)),
            full((64, 96)), full((1, 96)),
            full((216, 64)), full((1, 64)),
            full((256, 120)), full((1, 120)),
            full((120, 84)), full((1, 84)),
            full((84, 128)), full((1, 128)),
        ],
        out_specs=pl.BlockSpec((_NBLK, 128), lambda i: (i, 0)),
        compiler_params=pltpu.CompilerParams(
            dimension_semantics=("parallel",)),
    )(x4, w1blk, b1blk, w2blk, b2blk, wf1, bf1, wf2, bf2, wf3, bf3)
    return out[:n, :10]


# speedup vs baseline: 7.8917x; 3.5136x over previous
import jax
import jax.numpy as jnp
from jax.experimental import pallas as pl
from jax.experimental.pallas import tpu as pltpu

_NBLK = 256


def _lenet_block(x4_ref, w1_ref, b1_ref, w2_ref, b2_ref,
                 wf1_ref, bf1_ref, wf2_ref, bf2_ref, wf3_ref, bf3_ref,
                 out_ref):
    nb = x4_ref.shape[3]
    x4 = x4_ref[...]

    p1 = jnp.concatenate([
        jnp.concatenate([x4[u + ph2, v + pw2]
                         for ph2 in range(6) for pw2 in range(6)], axis=1)
        for u in range(2) for v in range(2)], axis=0)

    y1 = jnp.dot(w1_ref[...], p1,
                 preferred_element_type=jnp.float32) + b1_ref[...]
    m1 = y1.reshape(4, 32, 36 * nb)
    s1 = jnp.maximum(jnp.maximum(jnp.maximum(m1[0], m1[1]),
                                 jnp.maximum(m1[2], m1[3])), 0.0)

    p2 = jnp.concatenate([
        jnp.concatenate(
            [s1[:, ((u + qh) * 6 + (v + qw)) * nb:((u + qh) * 6 + (v + qw) + 1) * nb]
             for qh in range(4) for qw in range(4)], axis=1)
        for u in range(3) for v in range(3)], axis=0)

    y2 = jnp.dot(w2_ref[...], p2,
                 preferred_element_type=jnp.float32) + b2_ref[...]
    m2 = y2.reshape(4, 16, 16 * nb)
    s2 = jnp.maximum(jnp.maximum(jnp.maximum(m2[0], m2[1]),
                                 jnp.maximum(m2[2], m2[3])), 0.0)

    feats = jnp.concatenate([s2[:, s * nb:(s + 1) * nb] for s in range(16)],
                            axis=0)

    h1 = jnp.maximum(
        jnp.dot(wf1_ref[...], feats,
                preferred_element_type=jnp.float32) + bf1_ref[...], 0.0)
    h2 = jnp.maximum(
        jnp.dot(wf2_ref[...], h1,
                preferred_element_type=jnp.float32) + bf2_ref[...], 0.0)
    out_ref[...] = (
        jnp.dot(wf3_ref[...], h2,
                preferred_element_type=jnp.float32) + bf3_ref[...])


def kernel(x_nchw, w1blk, b1blk, w2blk, b2blk, wf1, bf1, wf2, bf2, wf3, bf3):
    n = x_nchw.shape[0]
    nb = max(_NBLK, ((n + _NBLK - 1) // _NBLK) * _NBLK)
    x = x_nchw.astype(jnp.float32).reshape(n, 28, 28)
    if nb != n:
        x = jnp.pad(x, ((0, nb - n), (0, 0), (0, 0)))
    x4 = jnp.transpose(x.reshape(nb, 7, 4, 7, 4),
                       (1, 3, 2, 4, 0)).reshape(7, 7, 16, nb)

    w1g = jnp.pad(w1blk.reshape(64, 4, 4, 6).transpose(2, 1, 3, 0)
                  .reshape(4, 24, 64), ((0, 0), (0, 8), (0, 0)))
    w1t = w1g.reshape(128, 64)
    b1t = jnp.pad(b1blk.reshape(4, 4, 6).transpose(1, 0, 2).reshape(4, 24),
                  ((0, 0), (0, 8))).reshape(128, 1)
    w2t = jnp.pad(w2blk.reshape(9, 24, 64), ((0, 0), (0, 8), (0, 0))) \
             .reshape(288, 64).T
    b2t = b2blk.reshape(64, 1)
    wf1t = wf1.T
    bf1t = bf1.reshape(120, 1)
    wf2t = wf2.T
    bf2t = bf2.reshape(84, 1)
    wf3t = wf3.T
    bf3t = bf3.reshape(128, 1)

    def full(shape):
        return pl.BlockSpec(shape, lambda i, _s=shape: (0,) * len(_s))

    out = pl.pallas_call(
        _lenet_block,
        out_shape=jax.ShapeDtypeStruct((128, nb), jnp.float32),
        grid=(nb // _NBLK,),
        in_specs=[
            pl.BlockSpec((7, 7, 16, _NBLK), lambda i: (0, 0, 0, i)),
            full((128, 64)), full((128, 1)),
            full((64, 288)), full((64, 1)),
            full((120, 256)), full((120, 1)),
            full((84, 120)), full((84, 1)),
            full((128, 84)), full((128, 1)),
        ],
        out_specs=pl.BlockSpec((128, _NBLK), lambda i: (0, i)),
        compiler_params=pltpu.CompilerParams(
            dimension_semantics=("parallel",)),
    )(x4, w1t, b1t, w2t, b2t, wf1t, bf1t, wf2t, bf2t, wf3t, bf3t)
    return out.T[:n, :10]


# speedup vs baseline: 8.1591x; 1.0339x over previous
import jax
import jax.numpy as jnp
from jax.experimental import pallas as pl
from jax.experimental.pallas import tpu as pltpu

_NBLK = 256


def _lenet_block(x4_ref, w1_ref, b1_ref, w2_ref, b2_ref,
                 wf1_ref, bf1_ref, wf2_ref, bf2_ref, wf3_ref, bf3_ref,
                 out_ref):
    nb = x4_ref.shape[3]
    x4 = x4_ref[...]

    p1 = jnp.concatenate([
        jnp.concatenate([x4[u + ph2, v + pw2]
                         for ph2 in range(6) for pw2 in range(6)], axis=1)
        for u in range(2) for v in range(2)], axis=0)

    y1 = jnp.dot(w1_ref[...], p1,
                 preferred_element_type=jnp.float32) + b1_ref[...]
    m1 = y1.reshape(4, 32, 36 * nb)
    s1 = jnp.maximum(jnp.maximum(jnp.maximum(m1[0], m1[1]),
                                 jnp.maximum(m1[2], m1[3])),
                     0.0).astype(jnp.bfloat16)

    p2 = jnp.concatenate([
        jnp.concatenate(
            [s1[:, ((u + qh) * 6 + (v + qw)) * nb:((u + qh) * 6 + (v + qw) + 1) * nb]
             for qh in range(4) for qw in range(4)], axis=1)
        for u in range(3) for v in range(3)], axis=0)

    y2 = jnp.dot(w2_ref[...], p2,
                 preferred_element_type=jnp.float32) + b2_ref[...]
    m2 = y2.reshape(4, 16, 16 * nb)
    s2 = jnp.maximum(jnp.maximum(jnp.maximum(m2[0], m2[1]),
                                 jnp.maximum(m2[2], m2[3])),
                     0.0).astype(jnp.bfloat16)

    feats = jnp.concatenate([s2[:, s * nb:(s + 1) * nb] for s in range(16)],
                            axis=0)

    h1 = jnp.maximum(
        jnp.dot(wf1_ref[...], feats,
                preferred_element_type=jnp.float32) + bf1_ref[...],
        0.0).astype(jnp.bfloat16)
    h2 = jnp.maximum(
        jnp.dot(wf2_ref[...], h1,
                preferred_element_type=jnp.float32) + bf2_ref[...],
        0.0).astype(jnp.bfloat16)
    out_ref[...] = (
        jnp.dot(wf3_ref[...], h2,
                preferred_element_type=jnp.float32) + bf3_ref[...])


def kernel(x_nchw, w1blk, b1blk, w2blk, b2blk, wf1, bf1, wf2, bf2, wf3, bf3):
    n = x_nchw.shape[0]
    nb = max(_NBLK, ((n + _NBLK - 1) // _NBLK) * _NBLK)
    x = x_nchw.astype(jnp.bfloat16).reshape(n, 28, 28)
    if nb != n:
        x = jnp.pad(x, ((0, nb - n), (0, 0), (0, 0)))
    x4 = jnp.transpose(x.reshape(nb, 7, 4, 7, 4),
                       (1, 3, 2, 4, 0)).reshape(7, 7, 16, nb)

    w1g = jnp.pad(w1blk.reshape(64, 4, 4, 6).transpose(2, 1, 3, 0)
                  .reshape(4, 24, 64), ((0, 0), (0, 8), (0, 0)))
    w1t = w1g.reshape(128, 64).astype(jnp.bfloat16)
    b1t = jnp.pad(b1blk.reshape(4, 4, 6).transpose(1, 0, 2).reshape(4, 24),
                  ((0, 0), (0, 8))).reshape(128, 1)
    w2t = jnp.pad(w2blk.reshape(9, 24, 64), ((0, 0), (0, 8), (0, 0))) \
             .reshape(288, 64).T.astype(jnp.bfloat16)
    b2t = b2blk.reshape(64, 1)
    wf1t = wf1.T.astype(jnp.bfloat16)
    bf1t = bf1.reshape(120, 1)
    wf2t = wf2.T.astype(jnp.bfloat16)
    bf2t = bf2.reshape(84, 1)
    wf3t = wf3.T.astype(jnp.bfloat16)
    bf3t = bf3.reshape(128, 1)

    def full(shape):
        return pl.BlockSpec(shape, lambda i, _s=shape: (0,) * len(_s))

    out = pl.pallas_call(
        _lenet_block,
        out_shape=jax.ShapeDtypeStruct((128, nb), jnp.float32),
        grid=(nb // _NBLK,),
        in_specs=[
            pl.BlockSpec((7, 7, 16, _NBLK), lambda i: (0, 0, 0, i)),
            full((128, 64)), full((128, 1)),
            full((64, 288)), full((64, 1)),
            full((120, 256)), full((120, 1)),
            full((84, 120)), full((84, 1)),
            full((128, 84)), full((128, 1)),
        ],
        out_specs=pl.BlockSpec((128, _NBLK), lambda i: (0, i)),
        compiler_params=pltpu.CompilerParams(
            dimension_semantics=("parallel",)),
    )(x4, w1t, b1t, w2t, b2t, wf1t, bf1t, wf2t, bf2t, wf3t, bf3t)
    return out.T[:n, :10]
